# baseline (device time: 59323 ns/iter reference)
import jax
import jax.numpy as jnp
from jax import lax
from jax.experimental import pallas as pl
from jax.experimental.pallas import tpu as pltpu

N_DEV = 4
CH = 4
_DORDER = (1, 3, 2)
_WBUF = {1: 1, 3: 2, 2: 0}
_STEPS = [(c, d) for c in range(CH) for d in _DORDER]


def kernel(x, w_mat):
    m_full, k_per = x.shape
    k_full, n = w_mat.shape
    m_per = m_full // N_DEV
    c_rows = m_per // CH

    def body(x_hbm, w_hbm, out_ref, xs_ref, xb_ref, xv_ref, ss_ref, sv_ref,
             xmy_ref, wv_ref, amax_ref,
             send_sems, recv_sems, sc_send_sems, sc_recv_sems,
             ax_send_sems, ax_recv_sems, wdma_sems, xdma_sem, sdma_sems):
        my = lax.axis_index("i")

        def src_dev(d):
            return lax.rem(my - d + N_DEV, N_DEV)

        def stage_copy(k):
            c, d = _STEPS[k]
            peer = lax.rem(my + d, N_DEV)
            return pltpu.make_async_copy(
                x_hbm.at[pl.ds(peer * m_per + c * c_rows, c_rows), :],
                xs_ref.at[k % 2], sdma_sems.at[k % 2])

        barrier_sem = pltpu.get_barrier_semaphore()
        for d in range(1, N_DEV):
            peer = lax.rem(my + d, N_DEV)
            pl.semaphore_signal(barrier_sem, inc=1, device_id=(peer,),
                                device_id_type=pl.DeviceIdType.MESH)
        pl.semaphore_wait(barrier_sem, N_DEV - 1)

        sdmas = {0: stage_copy(0), 1: stage_copy(1)}
        sdmas[0].start()
        sdmas[1].start()
        xdma = pltpu.make_async_copy(
            x_hbm.at[pl.ds(my * m_per, m_per), :], xmy_ref, xdma_sem)
        xdma.start()
        w_own = pltpu.make_async_copy(
            w_hbm.at[pl.ds(my * k_per, k_per), :], wv_ref.at[0],
            wdma_sems.at[0])
        w_own.start()
        w_d = {}
        for d in (1, 3):
            w_d[d] = pltpu.make_async_copy(
                w_hbm.at[pl.ds(src_dev(d) * k_per, k_per), :],
                wv_ref.at[_WBUF[d]], wdma_sems.at[_WBUF[d]])
            w_d[d].start()

        sends = []
        for k, (c, d) in enumerate(_STEPS):
            peer = lax.rem(my + d, N_DEV)
            rows = pl.ds(c * c_rows, c_rows)
            sdmas[k].wait()
            xchunk = xs_ref[k % 2]
            ramax = jnp.maximum(
                jnp.max(jnp.abs(xchunk), axis=1, keepdims=True), 1e-30)
            xb_ref[d - 1, rows, :] = jnp.clip(
                jnp.round(xchunk * (127.0 / ramax)), -127.0, 127.0
            ).astype(jnp.int8)
            ss_ref[d - 1, c, :] = jnp.reshape(ramax / 127.0, (c_rows,))
            if k + 2 < len(_STEPS):
                sdmas[k + 2] = stage_copy(k + 2)
                sdmas[k + 2].start()
            rdma = pltpu.make_async_remote_copy(
                src_ref=xb_ref.at[d - 1, rows, :],
                dst_ref=xv_ref.at[d - 1, rows, :],
                send_sem=send_sems.at[d - 1, c],
                recv_sem=recv_sems.at[d - 1, c],
                device_id=(peer,),
                device_id_type=pl.DeviceIdType.MESH,
            )
            rdma.start()
            sends.append(rdma)
            sc = pltpu.make_async_remote_copy(
                src_ref=ss_ref.at[d - 1, c, :],
                dst_ref=sv_ref.at[d - 1, c, :],
                send_sem=sc_send_sems.at[d - 1, c],
                recv_sem=sc_recv_sems.at[d - 1, c],
                device_id=(peer,),
                device_id_type=pl.DeviceIdType.MESH,
            )
            sc.start()
            sends.append(sc)

        xdma.wait()
        w_own.wait()
        out_ref[...] = jnp.dot(xmy_ref[...], wv_ref[0],
                               preferred_element_type=jnp.float32)
        w_d[2] = pltpu.make_async_copy(
            w_hbm.at[pl.ds(src_dev(2) * k_per, k_per), :], wv_ref.at[0],
            wdma_sems.at[0])
        w_d[2].start()
        for d in (1, 3):
            w_d[d].wait()

        maxes = []
        for c in range(CH):
            rows = pl.ds(c * c_rows, c_rows)
            for d in _DORDER:
                if c == 0 and d == 2:
                    w_d[2].wait()
                recv = pltpu.make_async_remote_copy(
                    src_ref=xv_ref.at[d - 1, rows, :],
                    dst_ref=xv_ref.at[d - 1, rows, :],
                    send_sem=send_sems.at[d - 1, c],
                    recv_sem=recv_sems.at[d - 1, c],
                    device_id=(src_dev(d),),
                    device_id_type=pl.DeviceIdType.MESH,
                )
                recv.wait_recv()
                sc_recv = pltpu.make_async_remote_copy(
                    src_ref=sv_ref.at[d - 1, c, :],
                    dst_ref=sv_ref.at[d - 1, c, :],
                    send_sem=sc_send_sems.at[d - 1, c],
                    recv_sem=sc_recv_sems.at[d - 1, c],
                    device_id=(src_dev(d),),
                    device_id_type=pl.DeviceIdType.MESH,
                )
                sc_recv.wait_recv()
                rscale = jnp.reshape(sv_ref[d - 1, c, :], (c_rows, 1))
                out_ref[rows, :] += jnp.dot(
                    xv_ref[d - 1, rows, :].astype(jnp.float32),
                    wv_ref[_WBUF[d]],
                    preferred_element_type=jnp.float32,
                ) * rscale
            maxes.append(jnp.max(jnp.abs(out_ref[rows, :])))

        for rdma in sends:
            rdma.wait_send()

        local_amax = maxes[0]
        for m in maxes[1:]:
            local_amax = jnp.maximum(local_amax, m)
        amax_ref[pl.ds(my, 1), :] = jnp.full((1, 128), local_amax, jnp.float32)
        ax_sends = []
        for d in range(1, N_DEV):
            peer = lax.rem(my + d, N_DEV)
            rdma = pltpu.make_async_remote_copy(
                src_ref=amax_ref.at[pl.ds(my, 1), :],
                dst_ref=amax_ref.at[pl.ds(my, 1), :],
                send_sem=ax_send_sems.at[d - 1],
                recv_sem=ax_recv_sems.at[d - 1],
                device_id=(peer,),
                device_id_type=pl.DeviceIdType.MESH,
            )
            rdma.start()
            ax_sends.append(rdma)
        for d in range(1, N_DEV):
            recv = pltpu.make_async_remote_copy(
                src_ref=amax_ref.at[pl.ds(src_dev(d), 1), :],
                dst_ref=amax_ref.at[pl.ds(src_dev(d), 1), :],
                send_sem=ax_send_sems.at[d - 1],
                recv_sem=ax_recv_sems.at[d - 1],
                device_id=(src_dev(d),),
                device_id_type=pl.DeviceIdType.MESH,
            )
            recv.wait_recv()
        for rdma in ax_sends:
            rdma.wait_send()

        g_amax = jnp.max(amax_ref[...])
        scale = g_amax / 127.0
        inv_scale = 127.0 / g_amax
        q = jnp.clip(jnp.round(out_ref[...] * inv_scale), -127.0, 127.0)
        out_ref[...] = q * scale

    return pl.pallas_call(
        body,
        out_shape=jax.ShapeDtypeStruct((m_per, n), jnp.float32),
        in_specs=[
            pl.BlockSpec(memory_space=pl.ANY),
            pl.BlockSpec(memory_space=pl.ANY),
        ],
        out_specs=pl.BlockSpec(memory_space=pltpu.VMEM),
        scratch_shapes=[
            pltpu.VMEM((2, c_rows, k_per), jnp.float32),
            pltpu.VMEM((N_DEV - 1, m_per, k_per), jnp.int8),
            pltpu.VMEM((N_DEV - 1, m_per, k_per), jnp.int8),
            pltpu.VMEM((N_DEV - 1, CH, c_rows), jnp.float32),
            pltpu.VMEM((N_DEV - 1, CH, c_rows), jnp.float32),
            pltpu.VMEM((m_per, k_per), jnp.float32),
            pltpu.VMEM((3, k_per, n), jnp.float32),
            pltpu.VMEM((N_DEV, 128), jnp.float32),
            pltpu.SemaphoreType.DMA((N_DEV - 1, CH)),
            pltpu.SemaphoreType.DMA((N_DEV - 1, CH)),
            pltpu.SemaphoreType.DMA((N_DEV - 1, CH)),
            pltpu.SemaphoreType.DMA((N_DEV - 1, CH)),
            pltpu.SemaphoreType.DMA((N_DEV - 1,)),
            pltpu.SemaphoreType.DMA((N_DEV - 1,)),
            pltpu.SemaphoreType.DMA((3,)),
            pltpu.SemaphoreType.DMA,
            pltpu.SemaphoreType.DMA((2,)),
        ],
        compiler_params=pltpu.CompilerParams(
            collective_id=0,
            vmem_limit_bytes=60 * 1024 * 1024,
        ),
    )(x, w_mat)


# device time: 53445 ns/iter; 1.1100x vs baseline; 1.1100x over previous
import jax
import jax.numpy as jnp
from jax import lax
from jax.experimental import pallas as pl
from jax.experimental.pallas import tpu as pltpu

N_DEV = 4
CH = 2
_DORDER = (1, 3, 2)
_WBUF = {1: 1, 3: 2, 2: 0}
_STEPS = [(c, d) for c in range(CH) for d in _DORDER]


def kernel(x, w_mat):
    m_full, k_per = x.shape
    k_full, n = w_mat.shape
    m_per = m_full // N_DEV
    c_rows = m_per // CH

    def body(x_hbm, w_hbm, out_ref, xs_ref, xb_ref, xv_ref, ss_ref, sv_ref,
             xmy_ref, wv_ref, amax_ref,
             send_sems, recv_sems, sc_send_sems, sc_recv_sems,
             ax_send_sems, ax_recv_sems, wdma_sems, xdma_sem, sdma_sems):
        my = lax.axis_index("i")

        def src_dev(d):
            return lax.rem(my - d + N_DEV, N_DEV)

        def stage_copy(k):
            c, d = _STEPS[k]
            peer = lax.rem(my + d, N_DEV)
            return pltpu.make_async_copy(
                x_hbm.at[pl.ds(peer * m_per + c * c_rows, c_rows), :],
                xs_ref.at[k % 2], sdma_sems.at[k % 2])

        barrier_sem = pltpu.get_barrier_semaphore()
        for d in range(1, N_DEV):
            peer = lax.rem(my + d, N_DEV)
            pl.semaphore_signal(barrier_sem, inc=1, device_id=(peer,),
                                device_id_type=pl.DeviceIdType.MESH)
        pl.semaphore_wait(barrier_sem, N_DEV - 1)

        sdmas = {0: stage_copy(0), 1: stage_copy(1)}
        sdmas[0].start()
        sdmas[1].start()
        xdma = pltpu.make_async_copy(
            x_hbm.at[pl.ds(my * m_per, m_per), :], xmy_ref, xdma_sem)
        xdma.start()
        w_own = pltpu.make_async_copy(
            w_hbm.at[pl.ds(my * k_per, k_per), :], wv_ref.at[0],
            wdma_sems.at[0])
        w_own.start()
        w_d = {}
        for d in (1, 3):
            w_d[d] = pltpu.make_async_copy(
                w_hbm.at[pl.ds(src_dev(d) * k_per, k_per), :],
                wv_ref.at[_WBUF[d]], wdma_sems.at[_WBUF[d]])
            w_d[d].start()

        sends = []
        for k, (c, d) in enumerate(_STEPS):
            peer = lax.rem(my + d, N_DEV)
            rows = pl.ds(c * c_rows, c_rows)
            sdmas[k].wait()
            xchunk = xs_ref[k % 2]
            ramax = jnp.maximum(
                jnp.max(jnp.abs(xchunk), axis=1, keepdims=True), 1e-30)
            xb_ref[d - 1, rows, :] = jnp.clip(
                jnp.round(xchunk * (127.0 / ramax)), -127.0, 127.0
            ).astype(jnp.int8)
            ss_ref[d - 1, c, :] = jnp.reshape(ramax / 127.0, (c_rows,))
            if k + 2 < len(_STEPS):
                sdmas[k + 2] = stage_copy(k + 2)
                sdmas[k + 2].start()
            rdma = pltpu.make_async_remote_copy(
                src_ref=xb_ref.at[d - 1, rows, :],
                dst_ref=xv_ref.at[d - 1, rows, :],
                send_sem=send_sems.at[d - 1, c],
                recv_sem=recv_sems.at[d - 1, c],
                device_id=(peer,),
                device_id_type=pl.DeviceIdType.MESH,
            )
            rdma.start()
            sends.append(rdma)
            sc = pltpu.make_async_remote_copy(
                src_ref=ss_ref.at[d - 1, c, :],
                dst_ref=sv_ref.at[d - 1, c, :],
                send_sem=sc_send_sems.at[d - 1, c],
                recv_sem=sc_recv_sems.at[d - 1, c],
                device_id=(peer,),
                device_id_type=pl.DeviceIdType.MESH,
            )
            sc.start()
            sends.append(sc)

        xdma.wait()
        w_own.wait()
        out_ref[...] = jnp.dot(xmy_ref[...], wv_ref[0],
                               preferred_element_type=jnp.float32)
        w_d[2] = pltpu.make_async_copy(
            w_hbm.at[pl.ds(src_dev(2) * k_per, k_per), :], wv_ref.at[0],
            wdma_sems.at[0])
        w_d[2].start()
        for d in (1, 3):
            w_d[d].wait()

        maxes = []
        for c in range(CH):
            rows = pl.ds(c * c_rows, c_rows)
            for d in _DORDER:
                if c == 0 and d == 2:
                    w_d[2].wait()
                recv = pltpu.make_async_remote_copy(
                    src_ref=xv_ref.at[d - 1, rows, :],
                    dst_ref=xv_ref.at[d - 1, rows, :],
                    send_sem=send_sems.at[d - 1, c],
                    recv_sem=recv_sems.at[d - 1, c],
                    device_id=(src_dev(d),),
                    device_id_type=pl.DeviceIdType.MESH,
                )
                recv.wait_recv()
                sc_recv = pltpu.make_async_remote_copy(
                    src_ref=sv_ref.at[d - 1, c, :],
                    dst_ref=sv_ref.at[d - 1, c, :],
                    send_sem=sc_send_sems.at[d - 1, c],
                    recv_sem=sc_recv_sems.at[d - 1, c],
                    device_id=(src_dev(d),),
                    device_id_type=pl.DeviceIdType.MESH,
                )
                sc_recv.wait_recv()
                rscale = jnp.reshape(sv_ref[d - 1, c, :], (c_rows, 1))
                out_ref[rows, :] += jnp.dot(
                    xv_ref[d - 1, rows, :].astype(jnp.float32),
                    wv_ref[_WBUF[d]],
                    preferred_element_type=jnp.float32,
                ) * rscale
            maxes.append(jnp.max(jnp.abs(out_ref[rows, :])))

        for rdma in sends:
            rdma.wait_send()

        local_amax = maxes[0]
        for m in maxes[1:]:
            local_amax = jnp.maximum(local_amax, m)
        amax_ref[pl.ds(my, 1), :] = jnp.full((1, 128), local_amax, jnp.float32)
        ax_sends = []
        for d in range(1, N_DEV):
            peer = lax.rem(my + d, N_DEV)
            rdma = pltpu.make_async_remote_copy(
                src_ref=amax_ref.at[pl.ds(my, 1), :],
                dst_ref=amax_ref.at[pl.ds(my, 1), :],
                send_sem=ax_send_sems.at[d - 1],
                recv_sem=ax_recv_sems.at[d - 1],
                device_id=(peer,),
                device_id_type=pl.DeviceIdType.MESH,
            )
            rdma.start()
            ax_sends.append(rdma)
        for d in range(1, N_DEV):
            recv = pltpu.make_async_remote_copy(
                src_ref=amax_ref.at[pl.ds(src_dev(d), 1), :],
                dst_ref=amax_ref.at[pl.ds(src_dev(d), 1), :],
                send_sem=ax_send_sems.at[d - 1],
                recv_sem=ax_recv_sems.at[d - 1],
                device_id=(src_dev(d),),
                device_id_type=pl.DeviceIdType.MESH,
            )
            recv.wait_recv()
        for rdma in ax_sends:
            rdma.wait_send()

        g_amax = jnp.max(amax_ref[...])
        scale = g_amax / 127.0
        inv_scale = 127.0 / g_amax
        q = jnp.clip(jnp.round(out_ref[...] * inv_scale), -127.0, 127.0)
        out_ref[...] = q * scale

    return pl.pallas_call(
        body,
        out_shape=jax.ShapeDtypeStruct((m_per, n), jnp.float32),
        in_specs=[
            pl.BlockSpec(memory_space=pl.ANY),
            pl.BlockSpec(memory_space=pl.ANY),
        ],
        out_specs=pl.BlockSpec(memory_space=pltpu.VMEM),
        scratch_shapes=[
            pltpu.VMEM((2, c_rows, k_per), jnp.float32),
            pltpu.VMEM((N_DEV - 1, m_per, k_per), jnp.int8),
            pltpu.VMEM((N_DEV - 1, m_per, k_per), jnp.int8),
            pltpu.VMEM((N_DEV - 1, CH, c_rows), jnp.float32),
            pltpu.VMEM((N_DEV - 1, CH, c_rows), jnp.float32),
            pltpu.VMEM((m_per, k_per), jnp.float32),
            pltpu.VMEM((3, k_per, n), jnp.float32),
            pltpu.VMEM((N_DEV, 128), jnp.float32),
            pltpu.SemaphoreType.DMA((N_DEV - 1, CH)),
            pltpu.SemaphoreType.DMA((N_DEV - 1, CH)),
            pltpu.SemaphoreType.DMA((N_DEV - 1, CH)),
            pltpu.SemaphoreType.DMA((N_DEV - 1, CH)),
            pltpu.SemaphoreType.DMA((N_DEV - 1,)),
            pltpu.SemaphoreType.DMA((N_DEV - 1,)),
            pltpu.SemaphoreType.DMA((3,)),
            pltpu.SemaphoreType.DMA,
            pltpu.SemaphoreType.DMA((2,)),
        ],
        compiler_params=pltpu.CompilerParams(
            collective_id=0,
            vmem_limit_bytes=60 * 1024 * 1024,
        ),
    )(x, w_mat)
